# baseline (speedup 1.0000x reference)
import functools
import math

import jax
import jax.numpy as jnp
from jax import lax
from jax.experimental import pallas as pl
from jax.experimental.pallas import tpu as pltpu
from jax.experimental.pallas import tpu_sc as plsc

NW = 32
NSUB = 16
SLICE = 128
_F32 = jnp.float32


def _mm_body(x_ref, w_ref, b_ref, o_ref, *, act):
    y = jnp.dot(x_ref[...].astype(jnp.bfloat16), w_ref[...].astype(jnp.bfloat16), preferred_element_type=_F32) + b_ref[...]
    if act == "relu":
        y = jnp.maximum(y, 0.0)
    o_ref[...] = y


def _mm(x, w, b, act="none", bm=1000):
    m, k = x.shape
    n = w.shape[1]
    return pl.pallas_call(
        functools.partial(_mm_body, act=act),
        grid=(m // bm,),
        in_specs=[
            pl.BlockSpec((bm, k), lambda i: (i, 0)),
            pl.BlockSpec((k, n), lambda i: (0, 0)),
            pl.BlockSpec((1, n), lambda i: (0, 0)),
        ],
        out_specs=pl.BlockSpec((bm, n), lambda i: (i, 0)),
        out_shape=jax.ShapeDtypeStruct((m, n), _F32),
    )(x, w, b.reshape(1, n))


def _out_ln_body(x_ref, w_ref, b_ref, g_ref, b2_ref, o_ref):
    y = jnp.dot(x_ref[...].astype(jnp.bfloat16), w_ref[...].astype(jnp.bfloat16),
                preferred_element_type=_F32) + b_ref[...]
    mu = jnp.mean(y, -1, keepdims=True)
    var = jnp.mean((y - mu) ** 2, -1, keepdims=True)
    o_ref[...] = (y - mu) / jnp.sqrt(var + 1e-5) * g_ref[...] + b2_ref[...]


def _out_ln(x, w, b, g, b2, m, bm=1000):
    k = x.shape[1]
    n = w.shape[1]
    return pl.pallas_call(
        _out_ln_body,
        grid=(m // bm,),
        in_specs=[
            pl.BlockSpec((bm, k), lambda i: (i, 0)),
            pl.BlockSpec((k, n), lambda i: (0, 0)),
            pl.BlockSpec((1, n), lambda i: (0, 0)),
            pl.BlockSpec((1, n), lambda i: (0, 0)),
            pl.BlockSpec((1, n), lambda i: (0, 0)),
        ],
        out_specs=pl.BlockSpec((bm, n), lambda i: (i, 0)),
        out_shape=jax.ShapeDtypeStruct((m, n), _F32),
    )(x, w, b.reshape(1, n), g.reshape(1, n), b2.reshape(1, n))


def _lane_take(v, idx):
    dnums = lax.GatherDimensionNumbers(
        offset_dims=(), collapsed_slice_dims=(0,), start_index_map=(0,))
    return lax.gather(v, idx[:, None], dnums, (1,),
                      mode=lax.GatherScatterMode.PROMISE_IN_BOUNDS)


_DOT_C = 32


@functools.lru_cache(maxsize=None)
def _make_dots(d, e_pad, e_real, scale):
    ew = e_pad // NW
    assert ew % _DOT_C == 0
    c = _DOT_C
    mesh = plsc.VectorSubcoreMesh(core_axis_name="c", subcore_axis_name="s")

    def body(kt, qt, si, di, a_out,
             sv0, dv0, krows0, qrows0, av0, dbuf0, semk0, semq0,
             sv1, dv1, krows1, qrows1, av1, dbuf1, semk1, semq1):
        cid = lax.axis_index("c")
        sid = lax.axis_index("s")
        base = (sid * 2 + cid) * ew
        lanes = lax.iota(jnp.int32, 16)
        zero16 = jnp.zeros((16,), _F32)
        nv = d // 16
        bufs = ((sv0, dv0, krows0, qrows0, av0, dbuf0, semk0, semq0),
                (sv1, dv1, krows1, qrows1, av1, dbuf1, semk1, semq1))
        nchunks = ew // c

        def issue(ci, p):
            sv, dv, krows, qrows, _, _, semk, semq = bufs[p]
            off = base + ci * c
            pltpu.sync_copy(si.at[pl.ds(off, c)], sv)
            pltpu.sync_copy(di.at[pl.ds(off, c)], dv)
            pltpu.async_copy(kt.at[sv], krows, semk)
            pltpu.async_copy(qt.at[dv], qrows, semq)

        def compute(ci, p):
            sv, dv, krows, qrows, av, dbuf, semk, semq = bufs[p]
            off = base + ci * c
            pltpu.make_async_copy(kt.at[sv], krows, semk).wait()
            pltpu.make_async_copy(qt.at[dv], qrows, semq).wait()
            for g in range(c // 16):
                for i in range(16):
                    e = g * 16 + i
                    accs = [zero16] * 4
                    for j in range(nv):
                        accs[j % 4] = accs[j % 4] + (
                            krows[e, pl.ds(j * 16, 16)]
                            * qrows[e, pl.ds(j * 16, 16)])
                    acc = (accs[0] + accs[1]) + (accs[2] + accs[3])
                    for sh in (8, 4, 2, 1):
                        acc = acc + _lane_take(acc, lanes ^ sh)
                    dbuf[i] = acc
                dots = plsc.load_gather(dbuf, [lanes, lanes]) * scale
                a16 = 1.0 / (1.0 + jnp.exp(-dots))
                gidx = off + g * 16 + lanes
                a16 = jnp.where(gidx < e_real, a16, 0.0)
                av[pl.ds(g * 16, 16)] = a16
            pltpu.sync_copy(av, a_out.at[pl.ds(off, c)])

        issue(0, 0)

        def macro(m, _):
            ci = m * 2

            @pl.when(ci + 1 < nchunks)
            def _():
                issue(ci + 1, 1)
            compute(ci, 0)

            @pl.when(ci + 2 < nchunks)
            def _():
                issue(ci + 2, 0)

            @pl.when(ci + 1 < nchunks)
            def _():
                compute(ci + 1, 1)
            return 0

        lax.fori_loop(0, (nchunks + 1) // 2, macro, 0)

    return pl.kernel(
        body,
        out_type=jax.ShapeDtypeStruct((e_pad,), _F32),
        mesh=mesh,
        compiler_params=pltpu.CompilerParams(needs_layout_passes=False),
        scratch_types=[
            pltpu.VMEM((c,), jnp.int32),
            pltpu.VMEM((c,), jnp.int32),
            pltpu.VMEM((c, d), _F32),
            pltpu.VMEM((c, d), _F32),
            pltpu.VMEM((c,), _F32),
            pltpu.VMEM((16, 16), _F32),
            pltpu.SemaphoreType.DMA,
            pltpu.SemaphoreType.DMA,
        ] * 2,
    )


_SC_C = 80


@functools.lru_cache(maxsize=None)
def _make_scatter(n_nodes, ns, e_pad):
    ew = e_pad // NSUB
    assert ew % _SC_C == 0
    c = _SC_C
    zrows = 128
    n_pad = -(-n_nodes // (NSUB * zrows)) * (NSUB * zrows)
    rows_per = n_pad // NSUB
    mesh = plsc.VectorSubcoreMesh(core_axis_name="c", subcore_axis_name="s")

    def body(*refs):
        vs = refs[:ns]
        si, di, av_hbm, out = refs[ns:ns + 4]
        acc, zbuf, sv, dv, ac, vrows, msg, sem = refs[ns + 4:]
        cid = lax.axis_index("c")
        sid = lax.axis_index("s")

        z16 = jnp.zeros((16,), _F32)

        def zr(r, _):
            for cb in range(SLICE // 16):
                zbuf[r, pl.ds(cb * 16, 16)] = z16
            return 0

        lax.fori_loop(0, zrows, zr, 0)

        for sl in range(ns):
            @pl.when(sl % 2 == cid)
            def _():
                for k in range(rows_per // zrows):
                    pltpu.sync_copy(
                        zbuf, acc.at[pl.ds(sid * rows_per + k * zrows, zrows)])
                plsc.subcore_barrier()

                def chunk(ci, _):
                    off = sid * ew + ci * c
                    pltpu.sync_copy(si.at[pl.ds(off, c)], sv)
                    pltpu.sync_copy(di.at[pl.ds(off, c)], dv)
                    pltpu.sync_copy(av_hbm.at[pl.ds(off, c)], ac)
                    pltpu.async_copy(vs[sl].at[sv], vrows, sem).wait()

                    def edge(e, _):
                        a = plsc.load_gather(ac, [jnp.full((16,), e, jnp.int32)])
                        for cb in range(SLICE // 16):
                            msg[e, pl.ds(cb * 16, 16)] = (
                                vrows[e, pl.ds(cb * 16, 16)] * a)
                        return 0

                    lax.fori_loop(0, c, edge, 0)
                    pltpu.sync_copy(msg, acc.at[dv], add=True)
                    return 0

                lax.fori_loop(0, ew // c, chunk, 0)
                plsc.subcore_barrier()
                pltpu.sync_copy(acc.at[pl.ds(sid * rows_per, rows_per)],
                                out.at[pl.ds(sid * rows_per, rows_per),
                                       pl.ds(sl * SLICE, SLICE)])
                plsc.subcore_barrier()

    return pl.kernel(
        body,
        out_type=jax.ShapeDtypeStruct((n_pad, ns * SLICE), _F32),
        mesh=mesh,
        compiler_params=pltpu.CompilerParams(needs_layout_passes=False),
        scratch_types=[
            pltpu.VMEM_SHARED((n_pad, SLICE), _F32),
            pltpu.VMEM((zrows, SLICE), _F32),
            pltpu.VMEM((c,), jnp.int32),
            pltpu.VMEM((c,), jnp.int32),
            pltpu.VMEM((c,), _F32),
            pltpu.VMEM((c, SLICE), _F32),
            pltpu.VMEM((c, SLICE), _F32),
            pltpu.SemaphoreType.DMA,
        ],
    )


def _pad_idx(x, e_pad):
    return jnp.pad(x, (0, e_pad - x.shape[0]))


def kernel(x_user, x_transaction, edge_index_pays, edge_index_rev, params):
    p = params
    e = edge_index_pays.shape[1]
    unit = (NW * _DOT_C * NSUB * _SC_C) // math.gcd(NW * _DOT_C, NSUB * _SC_C)
    e_pad = ((e + unit - 1) // unit) * unit

    s_p = _pad_idx(edge_index_pays[0], e_pad)
    d_p = _pad_idx(edge_index_pays[1], e_pad)
    s_r = _pad_idx(edge_index_rev[0], e_pad)
    d_r = _pad_idx(edge_index_rev[1], e_pad)

    hu = _mm(x_user, p["proj"]["user"]["W"], p["proj"]["user"]["b"], act="relu")
    ht = _mm(x_transaction, p["proj"]["transaction"]["W"],
             p["proj"]["transaction"]["b"], act="relu")

    for lp in p["layers"]:
        dout = lp["O"]["W"].shape[0]
        scale = 1.0 / math.sqrt(dout // 8)
        ns = dout // SLICE

        wu = jnp.concatenate([lp["K"]["pays"]["W"], lp["V"]["pays"]["W"],
                              lp["Q"]["rev"]["W"]], axis=1)
        bu = jnp.concatenate([lp["K"]["pays"]["b"], lp["V"]["pays"]["b"],
                              lp["Q"]["rev"]["b"]])
        wt = jnp.concatenate([lp["Q"]["pays"]["W"], lp["K"]["rev"]["W"],
                              lp["V"]["rev"]["W"]], axis=1)
        bt = jnp.concatenate([lp["Q"]["pays"]["b"], lp["K"]["rev"]["b"],
                              lp["V"]["rev"]["b"]])
        u = _mm(hu, wu, bu)
        t = _mm(ht, wt, bt)
        k_p, v_p, q_r = u[:, :dout], u[:, dout:2 * dout], u[:, 2 * dout:]
        q_p, k_r, v_r = t[:, :dout], t[:, dout:2 * dout], t[:, 2 * dout:]

        dots = _make_dots(dout, e_pad, e, scale)
        a_p = dots(k_p, q_p, s_p, d_p)
        a_r = dots(k_r, q_r, s_r, d_r)

        vp_sl = [v_p[:, i * SLICE:(i + 1) * SLICE] for i in range(ns)]
        vr_sl = [v_r[:, i * SLICE:(i + 1) * SLICE] for i in range(ns)]
        scat_t = _make_scatter(x_transaction.shape[0], ns, e_pad)
        scat_u = _make_scatter(x_user.shape[0], ns, e_pad)
        ot_acc = scat_t(*vp_sl, s_p, d_p, a_p)
        ou_acc = scat_u(*vr_sl, s_r, d_r, a_r)

        ht = _out_ln(ot_acc, lp["O"]["W"], lp["O"]["b"],
                     lp["ln"]["g"], lp["ln"]["b"], x_transaction.shape[0])
        hu = _out_ln(ou_acc, lp["O"]["W"], lp["O"]["b"],
                     lp["ln"]["g"], lp["ln"]["b"], x_user.shape[0])

    wth = jnp.pad(p["txn_head"]["W"], ((0, 0), (0, 127)))
    bth = jnp.pad(p["txn_head"]["b"], (0, 127))
    wuh = jnp.pad(p["user_head"]["W"], ((0, 0), (0, 127)))
    buh = jnp.pad(p["user_head"]["b"], (0, 127))
    txn_logits = _mm(ht, wth, bth)[:, 0]
    user_logits = _mm(hu, wuh, buh)[:, 0]
    return hu, ht, txn_logits, user_logits


# speedup vs baseline: 1.4054x; 1.4054x over previous
import functools
import math

import jax
import jax.numpy as jnp
from jax import lax
from jax.experimental import pallas as pl
from jax.experimental.pallas import tpu as pltpu
from jax.experimental.pallas import tpu_sc as plsc

NW = 32
NSUB = 16
SLICE = 128
_F32 = jnp.float32


def _mm_body(x_ref, w_ref, b_ref, o_ref, *, act):
    y = jnp.dot(x_ref[...].astype(jnp.bfloat16), w_ref[...].astype(jnp.bfloat16), preferred_element_type=_F32) + b_ref[...]
    if act == "relu":
        y = jnp.maximum(y, 0.0)
    o_ref[...] = y


def _mm(x, w, b, act="none", bm=1000):
    m, k = x.shape
    n = w.shape[1]
    return pl.pallas_call(
        functools.partial(_mm_body, act=act),
        grid=(m // bm,),
        in_specs=[
            pl.BlockSpec((bm, k), lambda i: (i, 0)),
            pl.BlockSpec((k, n), lambda i: (0, 0)),
            pl.BlockSpec((1, n), lambda i: (0, 0)),
        ],
        out_specs=pl.BlockSpec((bm, n), lambda i: (i, 0)),
        out_shape=jax.ShapeDtypeStruct((m, n), _F32),
    )(x, w, b.reshape(1, n))


def _out_ln_body(x_ref, w_ref, b_ref, g_ref, b2_ref, o_ref):
    y = jnp.dot(x_ref[...].astype(jnp.bfloat16), w_ref[...].astype(jnp.bfloat16),
                preferred_element_type=_F32) + b_ref[...]
    mu = jnp.mean(y, -1, keepdims=True)
    var = jnp.mean((y - mu) ** 2, -1, keepdims=True)
    o_ref[...] = (y - mu) / jnp.sqrt(var + 1e-5) * g_ref[...] + b2_ref[...]


def _out_ln(x, w, b, g, b2, m, bm=1000):
    k = x.shape[1]
    n = w.shape[1]
    return pl.pallas_call(
        _out_ln_body,
        grid=(m // bm,),
        in_specs=[
            pl.BlockSpec((bm, k), lambda i: (i, 0)),
            pl.BlockSpec((k, n), lambda i: (0, 0)),
            pl.BlockSpec((1, n), lambda i: (0, 0)),
            pl.BlockSpec((1, n), lambda i: (0, 0)),
            pl.BlockSpec((1, n), lambda i: (0, 0)),
        ],
        out_specs=pl.BlockSpec((bm, n), lambda i: (i, 0)),
        out_shape=jax.ShapeDtypeStruct((m, n), _F32),
    )(x, w, b.reshape(1, n), g.reshape(1, n), b2.reshape(1, n))


def _lane_take(v, idx):
    dnums = lax.GatherDimensionNumbers(
        offset_dims=(), collapsed_slice_dims=(0,), start_index_map=(0,))
    return lax.gather(v, idx[:, None], dnums, (1,),
                      mode=lax.GatherScatterMode.PROMISE_IN_BOUNDS)


_DOT_C = 32


@functools.lru_cache(maxsize=None)
def _make_dots(d, e_pad, e_real, scale):
    ew = e_pad // NW
    assert ew % _DOT_C == 0
    c = _DOT_C
    mesh = plsc.VectorSubcoreMesh(core_axis_name="c", subcore_axis_name="s")

    def body(kt, qt, si, di, a_out,
             sv, dv, av, dbuf, krows0, qrows0, semk0, semq0,
             krows1, qrows1, semk1, semq1):
        cid = lax.axis_index("c")
        sid = lax.axis_index("s")
        base = (sid * 2 + cid) * ew
        lanes = lax.iota(jnp.int32, 16)
        zero16 = jnp.zeros((16,), _F32)
        nv = d // 16
        bufs = ((krows0, qrows0, semk0, semq0),
                (krows1, qrows1, semk1, semq1))
        nchunks = ew // c

        pltpu.sync_copy(si.at[pl.ds(base, ew)], sv)
        pltpu.sync_copy(di.at[pl.ds(base, ew)], dv)

        def issue(ci, p):
            krows, qrows, semk, semq = bufs[p]
            pltpu.async_copy(kt.at[sv.at[pl.ds(ci * c, c)]], krows, semk)
            pltpu.async_copy(qt.at[dv.at[pl.ds(ci * c, c)]], qrows, semq)

        def compute(ci, p):
            krows, qrows, semk, semq = bufs[p]
            pltpu.make_async_copy(kt.at[sv.at[pl.ds(0, c)]], krows, semk).wait()
            pltpu.make_async_copy(qt.at[dv.at[pl.ds(0, c)]], qrows, semq).wait()

            def group(g, _):
                def edge(i, _):
                    e = g * 16 + i
                    accs = [zero16] * 4
                    for j in range(nv):
                        accs[j % 4] = accs[j % 4] + (
                            krows[e, pl.ds(j * 16, 16)]
                            * qrows[e, pl.ds(j * 16, 16)])
                    acc = (accs[0] + accs[1]) + (accs[2] + accs[3])
                    for sh in (8, 4, 2, 1):
                        acc = acc + _lane_take(acc, lanes ^ sh)
                    dbuf[i] = acc
                    return 0

                lax.fori_loop(0, 16, edge, 0)
                dots = plsc.load_gather(dbuf, [lanes, lanes]) * scale
                a16 = 1.0 / (1.0 + jnp.exp(-dots))
                gidx = base + ci * c + g * 16 + lanes
                a16 = jnp.where(gidx < e_real, a16, 0.0)
                av[pl.ds(ci * c + g * 16, 16)] = a16
                return 0

            lax.fori_loop(0, c // 16, group, 0)

        issue(0, 0)

        def macro(m, _):
            ci = m * 2

            @pl.when(ci + 1 < nchunks)
            def _():
                issue(ci + 1, 1)
            compute(ci, 0)

            @pl.when(ci + 2 < nchunks)
            def _():
                issue(ci + 2, 0)

            @pl.when(ci + 1 < nchunks)
            def _():
                compute(ci + 1, 1)
            return 0

        lax.fori_loop(0, (nchunks + 1) // 2, macro, 0)
        pltpu.sync_copy(av, a_out.at[pl.ds(base, ew)])

    return pl.kernel(
        body,
        out_type=jax.ShapeDtypeStruct((e_pad,), _F32),
        mesh=mesh,
        compiler_params=pltpu.CompilerParams(needs_layout_passes=False),
        scratch_types=[
            pltpu.VMEM((ew,), jnp.int32),
            pltpu.VMEM((ew,), jnp.int32),
            pltpu.VMEM((ew,), _F32),
            pltpu.VMEM((16, 16), _F32),
        ] + [
            pltpu.VMEM((c, d), _F32),
            pltpu.VMEM((c, d), _F32),
            pltpu.SemaphoreType.DMA,
            pltpu.SemaphoreType.DMA,
        ] * 2,
    )


_SC_C = 80


@functools.lru_cache(maxsize=None)
def _make_scatter(n_nodes, ns, e_pad):
    ew = e_pad // NSUB
    assert ew % _SC_C == 0
    c = _SC_C
    zrows = 128
    n_pad = -(-n_nodes // (NSUB * zrows)) * (NSUB * zrows)
    rows_per = n_pad // NSUB
    mesh = plsc.VectorSubcoreMesh(core_axis_name="c", subcore_axis_name="s")

    def body(*refs):
        vs = refs[:ns]
        si, di, av_hbm, out = refs[ns:ns + 4]
        acc, zbuf, sv, dv, ac, vrows, msg, sem = refs[ns + 4:]
        cid = lax.axis_index("c")
        sid = lax.axis_index("s")

        z16 = jnp.zeros((16,), _F32)

        def zr(r, _):
            for cb in range(SLICE // 16):
                zbuf[r, pl.ds(cb * 16, 16)] = z16
            return 0

        lax.fori_loop(0, zrows, zr, 0)

        for sl in range(ns):
            @pl.when(sl % 2 == cid)
            def _():
                for k in range(rows_per // zrows):
                    pltpu.sync_copy(
                        zbuf, acc.at[pl.ds(sid * rows_per + k * zrows, zrows)])
                plsc.subcore_barrier()

                def chunk(ci, _):
                    off = sid * ew + ci * c
                    pltpu.sync_copy(si.at[pl.ds(off, c)], sv)
                    pltpu.sync_copy(di.at[pl.ds(off, c)], dv)
                    pltpu.sync_copy(av_hbm.at[pl.ds(off, c)], ac)
                    pltpu.async_copy(vs[sl].at[sv], vrows, sem).wait()

                    def edge(e, _):
                        a = plsc.load_gather(ac, [jnp.full((16,), e, jnp.int32)])
                        for cb in range(SLICE // 16):
                            msg[e, pl.ds(cb * 16, 16)] = (
                                vrows[e, pl.ds(cb * 16, 16)] * a)
                        return 0

                    lax.fori_loop(0, c, edge, 0)
                    pltpu.sync_copy(msg, acc.at[dv], add=True)
                    return 0

                lax.fori_loop(0, ew // c, chunk, 0)
                plsc.subcore_barrier()
                pltpu.sync_copy(acc.at[pl.ds(sid * rows_per, rows_per)],
                                out.at[pl.ds(sid * rows_per, rows_per),
                                       pl.ds(sl * SLICE, SLICE)])
                plsc.subcore_barrier()

    return pl.kernel(
        body,
        out_type=jax.ShapeDtypeStruct((n_pad, ns * SLICE), _F32),
        mesh=mesh,
        compiler_params=pltpu.CompilerParams(needs_layout_passes=False),
        scratch_types=[
            pltpu.VMEM_SHARED((n_pad, SLICE), _F32),
            pltpu.VMEM((zrows, SLICE), _F32),
            pltpu.VMEM((c,), jnp.int32),
            pltpu.VMEM((c,), jnp.int32),
            pltpu.VMEM((c,), _F32),
            pltpu.VMEM((c, SLICE), _F32),
            pltpu.VMEM((c, SLICE), _F32),
            pltpu.SemaphoreType.DMA,
        ],
    )


def _pad_idx(x, e_pad):
    return jnp.pad(x, (0, e_pad - x.shape[0]))


def kernel(x_user, x_transaction, edge_index_pays, edge_index_rev, params):
    p = params
    e = edge_index_pays.shape[1]
    unit = (NW * _DOT_C * NSUB * _SC_C) // math.gcd(NW * _DOT_C, NSUB * _SC_C)
    e_pad = ((e + unit - 1) // unit) * unit

    s_p = _pad_idx(edge_index_pays[0], e_pad)
    d_p = _pad_idx(edge_index_pays[1], e_pad)
    s_r = _pad_idx(edge_index_rev[0], e_pad)
    d_r = _pad_idx(edge_index_rev[1], e_pad)

    hu = _mm(x_user, p["proj"]["user"]["W"], p["proj"]["user"]["b"], act="relu")
    ht = _mm(x_transaction, p["proj"]["transaction"]["W"],
             p["proj"]["transaction"]["b"], act="relu")

    for lp in p["layers"]:
        dout = lp["O"]["W"].shape[0]
        scale = 1.0 / math.sqrt(dout // 8)
        ns = dout // SLICE

        wu = jnp.concatenate([lp["K"]["pays"]["W"], lp["V"]["pays"]["W"],
                              lp["Q"]["rev"]["W"]], axis=1)
        bu = jnp.concatenate([lp["K"]["pays"]["b"], lp["V"]["pays"]["b"],
                              lp["Q"]["rev"]["b"]])
        wt = jnp.concatenate([lp["Q"]["pays"]["W"], lp["K"]["rev"]["W"],
                              lp["V"]["rev"]["W"]], axis=1)
        bt = jnp.concatenate([lp["Q"]["pays"]["b"], lp["K"]["rev"]["b"],
                              lp["V"]["rev"]["b"]])
        u = _mm(hu, wu, bu)
        t = _mm(ht, wt, bt)
        k_p, v_p, q_r = u[:, :dout], u[:, dout:2 * dout], u[:, 2 * dout:]
        q_p, k_r, v_r = t[:, :dout], t[:, dout:2 * dout], t[:, 2 * dout:]

        dots = _make_dots(dout, e_pad, e, scale)
        a_p = dots(k_p, q_p, s_p, d_p)
        a_r = dots(k_r, q_r, s_r, d_r)

        vp_sl = [v_p[:, i * SLICE:(i + 1) * SLICE] for i in range(ns)]
        vr_sl = [v_r[:, i * SLICE:(i + 1) * SLICE] for i in range(ns)]
        scat_t = _make_scatter(x_transaction.shape[0], ns, e_pad)
        scat_u = _make_scatter(x_user.shape[0], ns, e_pad)
        ot_acc = scat_t(*vp_sl, s_p, d_p, a_p)
        ou_acc = scat_u(*vr_sl, s_r, d_r, a_r)

        ht = _out_ln(ot_acc, lp["O"]["W"], lp["O"]["b"],
                     lp["ln"]["g"], lp["ln"]["b"], x_transaction.shape[0])
        hu = _out_ln(ou_acc, lp["O"]["W"], lp["O"]["b"],
                     lp["ln"]["g"], lp["ln"]["b"], x_user.shape[0])

    wth = jnp.pad(p["txn_head"]["W"], ((0, 0), (0, 127)))
    bth = jnp.pad(p["txn_head"]["b"], (0, 127))
    wuh = jnp.pad(p["user_head"]["W"], ((0, 0), (0, 127)))
    buh = jnp.pad(p["user_head"]["b"], (0, 127))
    txn_logits = _mm(ht, wth, bth)[:, 0]
    user_logits = _mm(hu, wuh, buh)[:, 0]
    return hu, ht, txn_logits, user_logits
